# baseline (speedup 1.0000x reference)
import jax
import jax.numpy as jnp
from jax.experimental import pallas as pl
from jax.experimental.pallas import tpu as pltpu

K = 1024
D = 64
BETA = 0.25
HW = 1024
B = 16
BLK_B = 2
GRID = B // BLK_B


def _vq_kernel(x_ref, emb_ref, q_ref, idx_ref, loss_ref):
    emb = emb_ref[...]
    e2 = jnp.sum(emb * emb, axis=1, keepdims=True)
    partial = jnp.zeros((1, 1), jnp.float32)
    for b in range(BLK_B):
        x = x_ref[b]
        x2 = jnp.sum(x * x, axis=0, keepdims=True)
        m = jax.lax.dot_general(
            emb, x, (((1,), (0,)), ((), ())),
            preferred_element_type=jnp.float32,
        )
        dist = (x2 + e2) - 2.0 * m

        idx = jnp.argmin(dist, axis=0)
        kiota = jax.lax.broadcasted_iota(jnp.int32, (K, HW), 0)
        onehot = (kiota == idx[None, :]).astype(jnp.float32)
        quant = jax.lax.dot_general(
            emb, onehot, (((0,), (0,)), ((), ())),
            preferred_element_type=jnp.float32,
        )

        q_ref[b] = quant
        idx_ref[b] = idx.reshape(1, HW)

        diff = quant - x
        partial = partial + jnp.sum(diff * diff).reshape(1, 1)

    loss_ref[0] = partial


@jax.jit
def kernel(enc_pred, embeddings):
    x3 = enc_pred.reshape(B, D, HW)
    q, idx, loss_parts = pl.pallas_call(
        _vq_kernel,
        grid=(GRID,),
        in_specs=[
            pl.BlockSpec((BLK_B, D, HW), lambda s: (s, 0, 0)),
            pl.BlockSpec((K, D), lambda s: (0, 0)),
        ],
        out_specs=[
            pl.BlockSpec((BLK_B, D, HW), lambda s: (s, 0, 0)),
            pl.BlockSpec((BLK_B, 1, HW), lambda s: (s, 0, 0)),
            pl.BlockSpec((1, 1, 1), lambda s: (s, 0, 0)),
        ],
        out_shape=[
            jax.ShapeDtypeStruct((B, D, HW), jnp.float32),
            jax.ShapeDtypeStruct((B, 1, HW), jnp.int32),
            jax.ShapeDtypeStruct((GRID, 1, 1), jnp.float32),
        ],
        compiler_params=pltpu.CompilerParams(
            dimension_semantics=("parallel",),
        ),
    )(x3, embeddings)
    quantized_out = q.reshape(B, D, 32, 32)
    indices_out = idx.reshape(B, 1, 32, 32)
    loss = jnp.sum(loss_parts) * (BETA / (B * HW * D))
    return (quantized_out, loss, indices_out)


# speedup vs baseline: 1.0705x; 1.0705x over previous
import jax
import jax.numpy as jnp
from jax.experimental import pallas as pl
from jax.experimental.pallas import tpu as pltpu

K = 1024
D = 64
BETA = 0.25
HW = 1024
B = 16
BLK_B = 8
GRID = B // BLK_B


def _vq_kernel(x_ref, emb_ref, q_ref, idx_ref, loss_ref, e2_ref):
    step = pl.program_id(0)
    emb = emb_ref[...]

    @pl.when(step == 0)
    def _prep():
        e2_ref[...] = jnp.sum(emb * emb, axis=1, keepdims=True)

    e2 = e2_ref[...]
    partial = jnp.zeros((1, 1), jnp.float32)
    for b in range(BLK_B):
        x = x_ref[b]
        x2 = jnp.sum(x * x, axis=0, keepdims=True)
        m = jax.lax.dot_general(
            emb, x, (((1,), (0,)), ((), ())),
            preferred_element_type=jnp.float32,
        )
        dist = (x2 + e2) - 2.0 * m

        idx = jnp.argmin(dist, axis=0)
        kiota = jax.lax.broadcasted_iota(jnp.int32, (K, HW), 0)
        onehot = (kiota == idx[None, :]).astype(jnp.float32)
        quant = jax.lax.dot_general(
            emb, onehot, (((0,), (0,)), ((), ())),
            preferred_element_type=jnp.float32,
        )

        q_ref[b] = quant
        idx_ref[b] = idx.reshape(1, HW)

        diff = quant - x
        partial = partial + jnp.sum(diff * diff).reshape(1, 1)

    @pl.when(step == 0)
    def _init():
        loss_ref[...] = partial

    @pl.when(step != 0)
    def _acc():
        loss_ref[...] += partial

    @pl.when(step == GRID - 1)
    def _scale():
        loss_ref[...] *= BETA / (B * HW * D)


@jax.jit
def kernel(enc_pred, embeddings):
    x3 = enc_pred.reshape(B, D, HW)
    q, idx, loss = pl.pallas_call(
        _vq_kernel,
        grid=(GRID,),
        in_specs=[
            pl.BlockSpec((BLK_B, D, HW), lambda s: (s, 0, 0)),
            pl.BlockSpec((K, D), lambda s: (0, 0)),
        ],
        out_specs=[
            pl.BlockSpec((BLK_B, D, HW), lambda s: (s, 0, 0)),
            pl.BlockSpec((BLK_B, 1, HW), lambda s: (s, 0, 0)),
            pl.BlockSpec((1, 1), lambda s: (0, 0)),
        ],
        out_shape=[
            jax.ShapeDtypeStruct((B, D, HW), jnp.float32),
            jax.ShapeDtypeStruct((B, 1, HW), jnp.int32),
            jax.ShapeDtypeStruct((1, 1), jnp.float32),
        ],
        scratch_shapes=[pltpu.VMEM((K, 1), jnp.float32)],
    )(x3, embeddings)
    quantized_out = q.reshape(B, D, 32, 32)
    indices_out = idx.reshape(B, 1, 32, 32)
    return (quantized_out, loss.reshape(()), indices_out)
